# Initial kernel scaffold; baseline (speedup 1.0000x reference)
import jax
import jax.numpy as jnp
from jax.experimental import pallas as pl


def kernel(node_feats, edge_index, edge_feats, Ws0, Wn0, b0, Ws1, Wn1, b1, Ws2, Wn2, b2, W1, bm1, W2, bm2):
    raise NotImplementedError("write your pallas kernel here")


# speedup vs baseline: 3.0932x; 3.0932x over previous
import functools

import jax
import jax.numpy as jnp
from jax import lax
from jax.experimental import pallas as pl
from jax.experimental.pallas import tpu as pltpu
from jax.experimental.pallas import tpu_sc as plsc

N = 10000
E = 320000
D = 128
DE = 4
C = 2

NC = 2
NS = 16
NW = NC * NS
EPW = E // NW
CH = 80
NCHUNK = EPW // CH
ROWS_PER_TILE = 624
TAIL_ROWS = N - NS * ROWS_PER_TILE
TAIL_OFF = NS * ROWS_PER_TILE


def _tile_table_copy(s, mk_src, mk_dst):
    r0 = s * ROWS_PER_TILE
    pltpu.sync_copy(mk_src(r0, ROWS_PER_TILE), mk_dst(r0, ROWS_PER_TILE))

    @pl.when(s == NS - 1)
    def _():
        pltpu.sync_copy(mk_src(TAIL_OFF, TAIL_ROWS), mk_dst(TAIL_OFF, TAIL_ROWS))

_mesh = plsc.VectorSubcoreMesh(
    core_axis_name="c", subcore_axis_name="s", num_cores=NC, num_subcores=NS)


def _seg_sum_body(p_hbm, src_hbm, dst_hbm, z128_hbm,
                  agg_out, sidx, didx, rows, agg_sh, sem):
    c = lax.axis_index("c")
    s = lax.axis_index("s")
    wid = c * NS + s
    base = wid * EPW

    _tile_table_copy(s, lambda o, n: z128_hbm.at[pl.ds(o, n)],
                     lambda o, n: agg_sh.at[pl.ds(o, n)])
    plsc.subcore_barrier()

    def chunk(j, carry):
        off = base + j * CH
        pltpu.sync_copy(src_hbm.at[pl.ds(off, CH)], sidx)
        pltpu.sync_copy(dst_hbm.at[pl.ds(off, CH)], didx)
        pltpu.async_copy(p_hbm.at[sidx], rows, sem).wait()
        pltpu.sync_copy(rows, agg_sh.at[didx], add=True)
        return carry

    lax.fori_loop(0, NCHUNK, chunk, 0)
    plsc.subcore_barrier()

    _tile_table_copy(s, lambda o, n: agg_sh.at[pl.ds(o, n)],
                     lambda o, n: agg_out.at[c, pl.ds(o, n)])


_seg_sum = pl.kernel(
    _seg_sum_body,
    out_type=jax.ShapeDtypeStruct((NC, N, D), jnp.float32),
    mesh=_mesh,
    scratch_types=[
        pltpu.VMEM((CH,), jnp.int32),
        pltpu.VMEM((CH,), jnp.int32),
        pltpu.VMEM((CH, D), jnp.float32),
        pltpu.VMEM_SHARED((N, D), jnp.float32),
        pltpu.SemaphoreType.DMA,
    ])


def _deg_hist_body(ones_hbm, dst_hbm, z128_hbm, deg_out, didx, rows, deg_sh, sem):
    c = lax.axis_index("c")
    s = lax.axis_index("s")
    wid = c * NS + s
    base = wid * EPW

    _tile_table_copy(s, lambda o, n: z128_hbm.at[pl.ds(o, n)],
                     lambda o, n: deg_sh.at[pl.ds(o, n)])
    pltpu.sync_copy(ones_hbm, rows)
    plsc.subcore_barrier()

    def chunk(j, carry):
        off = base + j * CH
        pltpu.sync_copy(dst_hbm.at[pl.ds(off, CH)], didx)
        pltpu.sync_copy(rows, deg_sh.at[didx], add=True)
        return carry

    lax.fori_loop(0, NCHUNK, chunk, 0)
    plsc.subcore_barrier()

    _tile_table_copy(s, lambda o, n: deg_sh.at[pl.ds(o, n)],
                     lambda o, n: deg_out.at[c, pl.ds(o, n)])


_deg_hist = pl.kernel(
    _deg_hist_body,
    out_type=jax.ShapeDtypeStruct((NC, N, D), jnp.float32),
    mesh=_mesh,
    scratch_types=[
        pltpu.VMEM((CH,), jnp.int32),
        pltpu.VMEM((CH, D), jnp.float32),
        pltpu.VMEM_SHARED((N, D), jnp.float32),
        pltpu.SemaphoreType.DMA,
    ])


def _final_gather_body(sp_hbm, dp_hbm, src_hbm, dst_hbm, us_out, ud_out,
                       sidx, didx, rows_s, rows_d, sem):
    c = lax.axis_index("c")
    s = lax.axis_index("s")
    wid = c * NS + s
    base = wid * EPW

    def chunk(j, carry):
        off = base + j * CH
        pltpu.sync_copy(src_hbm.at[pl.ds(off, CH)], sidx)
        pltpu.sync_copy(dst_hbm.at[pl.ds(off, CH)], didx)
        pltpu.async_copy(sp_hbm.at[sidx], rows_s, sem).wait()
        pltpu.async_copy(dp_hbm.at[didx], rows_d, sem).wait()
        pltpu.sync_copy(rows_s, us_out.at[pl.ds(off, CH)])
        pltpu.sync_copy(rows_d, ud_out.at[pl.ds(off, CH)])
        return carry

    lax.fori_loop(0, NCHUNK, chunk, 0)


_final_gather = pl.kernel(
    _final_gather_body,
    out_type=(jax.ShapeDtypeStruct((E, D), jnp.float32),
              jax.ShapeDtypeStruct((E, D), jnp.float32)),
    mesh=_mesh,
    scratch_types=[
        pltpu.VMEM((CH,), jnp.int32),
        pltpu.VMEM((CH,), jnp.int32),
        pltpu.VMEM((CH, D), jnp.float32),
        pltpu.VMEM((CH, D), jnp.float32),
        pltpu.SemaphoreType.DMA,
    ])


def _mm_body(h_ref, w_ref, o_ref):
    o_ref[...] = jnp.dot(h_ref[...], w_ref[...],
                         preferred_element_type=jnp.float32)


_mm = pl.pallas_call(_mm_body, out_shape=jax.ShapeDtypeStruct((N, D), jnp.float32))


def _combine_body(h_ref, agg_ref, deg_ref, ws_ref, b_ref, wn_ref, h_out, p_out):
    a = agg_ref[0] + agg_ref[1]
    dcol = deg_ref[0, :, 0:1] + deg_ref[1, :, 0:1]
    inv = 1.0 / jnp.maximum(dcol, 1.0)
    x = jnp.dot(h_ref[...], ws_ref[...], preferred_element_type=jnp.float32)
    x = x + a * inv + b_ref[...]
    hn = jnp.where(x > 0, x, 0.01 * x)
    h_out[...] = hn
    p_out[...] = jnp.dot(hn, wn_ref[...], preferred_element_type=jnp.float32)


_combine = pl.pallas_call(
    _combine_body,
    out_shape=(jax.ShapeDtypeStruct((N, D), jnp.float32),
               jax.ShapeDtypeStruct((N, D), jnp.float32)))


def _combine3_body(h_ref, agg_ref, deg_ref, ws_ref, b_ref, w1a_ref, w1b_ref,
                   bm1_ref, sp_out, dp_out):
    a = agg_ref[0] + agg_ref[1]
    dcol = deg_ref[0, :, 0:1] + deg_ref[1, :, 0:1]
    inv = 1.0 / jnp.maximum(dcol, 1.0)
    x = jnp.dot(h_ref[...], ws_ref[...], preferred_element_type=jnp.float32)
    x = x + a * inv + b_ref[...]
    hn = jnp.where(x > 0, x, 0.01 * x)
    sp_out[...] = jnp.dot(hn, w1a_ref[...],
                          preferred_element_type=jnp.float32) + bm1_ref[...]
    dp_out[...] = jnp.dot(hn, w1b_ref[...], preferred_element_type=jnp.float32)


_combine3 = pl.pallas_call(
    _combine3_body,
    out_shape=(jax.ShapeDtypeStruct((N, D), jnp.float32),
               jax.ShapeDtypeStruct((N, D), jnp.float32)))

EBLK = 2000


def _edge_mlp_body(us_ref, ud_ref, ef_ref, w1e_ref, w2_ref, bm2_ref, o_ref):
    z = us_ref[...] + ud_ref[...] + jnp.dot(
        ef_ref[...], w1e_ref[...], preferred_element_type=jnp.float32)
    z = jnp.maximum(z, 0.0)
    o_ref[...] = jnp.dot(z, w2_ref[...],
                         preferred_element_type=jnp.float32) + bm2_ref[...]


_edge_mlp = pl.pallas_call(
    _edge_mlp_body,
    grid=(E // EBLK,),
    in_specs=[
        pl.BlockSpec((EBLK, D), lambda i: (i, 0)),
        pl.BlockSpec((EBLK, D), lambda i: (i, 0)),
        pl.BlockSpec((EBLK, DE), lambda i: (i, 0)),
        pl.BlockSpec((DE, D), lambda i: (0, 0)),
        pl.BlockSpec((D, C), lambda i: (0, 0)),
        pl.BlockSpec((1, C), lambda i: (0, 0)),
    ],
    out_specs=pl.BlockSpec((EBLK, C), lambda i: (i, 0)),
    out_shape=jax.ShapeDtypeStruct((E, C), jnp.float32))


def kernel(node_feats, edge_index, edge_feats,
           Ws0, Wn0, b0, Ws1, Wn1, b1, Ws2, Wn2, b2,
           W1, bm1, W2, bm2):
    src = edge_index[0]
    dst = edge_index[1]
    z128 = jnp.zeros((N, D), jnp.float32)
    ones_ch = jnp.ones((CH, D), jnp.float32)

    deg = _deg_hist(ones_ch, dst, z128)
    p0 = _mm(node_feats, Wn0)
    agg0 = _seg_sum(p0, src, dst, z128)
    h1, p1 = _combine(node_feats, agg0, deg, Ws0, b0.reshape(1, D), Wn1)
    agg1 = _seg_sum(p1, src, dst, z128)
    h2, p2 = _combine(h1, agg1, deg, Ws1, b1.reshape(1, D), Wn2)
    agg2 = _seg_sum(p2, src, dst, z128)
    sp, dp = _combine3(h2, agg2, deg, Ws2, b2.reshape(1, D),
                       W1[:D], W1[D:2 * D], bm1.reshape(1, D))
    us, ud = _final_gather(sp, dp, src, dst)
    logits = _edge_mlp(us, ud, edge_feats, W1[2 * D:], W2, bm2.reshape(1, C))
    return logits
